# baseline (speedup 1.0000x reference)
import functools

import jax
import jax.numpy as jnp
from jax.experimental import pallas as pl
from jax.experimental.pallas import tpu as pltpu

_E_BLK = 2000
_N_BLK = 1000


def _stats0_body(xi_ref, xj_ref, st_ref):
    xi = xi_ref[...]
    d = xj_ref[...] - xi
    a = jnp.concatenate([xi, d], axis=1)

    @pl.when(pl.program_id(0) == 0)
    def _():
        st_ref[...] = jnp.zeros_like(st_ref)

    st_ref[...] += jnp.stack([jnp.sum(a, 0), jnp.sum(a * a, 0)], 0)


def _mm_first_body(xi_ref, xj_ref, a_ref, b_ref, c_ref, h_ref, st_ref):
    h = jnp.dot(xi_ref[...], a_ref[...], preferred_element_type=jnp.float32)
    h += jnp.dot(xj_ref[...], b_ref[...], preferred_element_type=jnp.float32)
    h = jnp.maximum(h + c_ref[...], 0.0)
    h_ref[...] = h

    @pl.when(pl.program_id(0) == 0)
    def _():
        st_ref[...] = jnp.zeros_like(st_ref)

    st_ref[...] += jnp.stack([jnp.sum(h, 0), jnp.sum(h * h, 0)], 0)


def _mm_mid_body(x_ref, w_ref, c_ref, h_ref, st_ref):
    h = jnp.dot(x_ref[...], w_ref[...], preferred_element_type=jnp.float32)
    h = jnp.maximum(h + c_ref[...], 0.0)
    h_ref[...] = h

    @pl.when(pl.program_id(0) == 0)
    def _():
        st_ref[...] = jnp.zeros_like(st_ref)

    st_ref[...] += jnp.stack([jnp.sum(h, 0), jnp.sum(h * h, 0)], 0)


def _finalize_body(raw_ref, s_ref, t_ref, o_ref):
    raw = raw_ref[...]
    val = raw * s_ref[...] + t_ref[...]
    o_ref[...] = jnp.where(raw > -jnp.inf, val, 0.0)


def _head_body(raw_ref, s_ref, t_ref, w1_ref, b1_ref, w2_ref, b2_ref,
               w3_ref, b3_ref, w4_ref, b4_ref, o_ref):
    raw = raw_ref[...]
    h = jnp.where(raw > -jnp.inf, raw * s_ref[...] + t_ref[...], 0.0)
    h = jnp.maximum(jnp.dot(h, w1_ref[...], preferred_element_type=jnp.float32)
                    + b1_ref[...], 0.0)
    h = jnp.maximum(jnp.dot(h, w2_ref[...], preferred_element_type=jnp.float32)
                    + b2_ref[...], 0.0)
    h = jnp.dot(h, w3_ref[...], preferred_element_type=jnp.float32) + b3_ref[...]
    h = jnp.dot(h, w4_ref[...], preferred_element_type=jnp.float32) + b4_ref[...]
    o_ref[...] = 1.0 / (1.0 + jnp.exp(-h))


def _row_spec(blk, f):
    return pl.BlockSpec((blk, f), lambda i: (i, 0))


def _full_spec(shape):
    return pl.BlockSpec(shape, lambda i: (0,) * len(shape))


def _stats0(xi, xj):
    e, f = xi.shape
    return pl.pallas_call(
        _stats0_body,
        grid=(e // _E_BLK,),
        in_specs=[_row_spec(_E_BLK, f), _row_spec(_E_BLK, f)],
        out_specs=_full_spec((2, 2 * f)),
        out_shape=jax.ShapeDtypeStruct((2, 2 * f), jnp.float32),
    )(xi, xj)


def _mm_first(xi, xj, a, b, c):
    e, fi = xi.shape
    fo = a.shape[1]
    return pl.pallas_call(
        _mm_first_body,
        grid=(e // _E_BLK,),
        in_specs=[_row_spec(_E_BLK, fi), _row_spec(_E_BLK, fi),
                  _full_spec((fi, fo)), _full_spec((fi, fo)),
                  _full_spec((1, fo))],
        out_specs=[_row_spec(_E_BLK, fo), _full_spec((2, fo))],
        out_shape=[jax.ShapeDtypeStruct((e, fo), jnp.float32),
                   jax.ShapeDtypeStruct((2, fo), jnp.float32)],
    )(xi, xj, a, b, c.reshape(1, fo))


def _mm_mid(x, w, c):
    e, fi = x.shape
    fo = w.shape[1]
    return pl.pallas_call(
        _mm_mid_body,
        grid=(e // _E_BLK,),
        in_specs=[_row_spec(_E_BLK, fi), _full_spec((fi, fo)),
                  _full_spec((1, fo))],
        out_specs=[_row_spec(_E_BLK, fo), _full_spec((2, fo))],
        out_shape=[jax.ShapeDtypeStruct((e, fo), jnp.float32),
                   jax.ShapeDtypeStruct((2, fo), jnp.float32)],
    )(x, w, c.reshape(1, fo))


def _finalize(raw, s, t):
    n, f = raw.shape
    blk = _N_BLK
    return pl.pallas_call(
        _finalize_body,
        grid=(n // blk,),
        in_specs=[_row_spec(blk, f), _full_spec((1, f)), _full_spec((1, f))],
        out_specs=_row_spec(blk, f),
        out_shape=jax.ShapeDtypeStruct((n, f), jnp.float32),
    )(raw, s.reshape(1, f), t.reshape(1, f))


def _head(raw, s, t, p):
    n, f = raw.shape
    blk = _N_BLK
    args = [raw, s.reshape(1, f), t.reshape(1, f),
            p['h_w1'].T, p['h_b1'].reshape(1, -1),
            p['h_w2'].T, p['h_b2'].reshape(1, -1),
            p['h_w3'].T, p['h_b3'].reshape(1, -1),
            p['h_w4'].T, p['h_b4'].reshape(1, -1)]
    in_specs = [_row_spec(blk, f)] + [_full_spec(a.shape) for a in args[1:]]
    return pl.pallas_call(
        _head_body,
        grid=(n // blk,),
        in_specs=in_specs,
        out_specs=_row_spec(blk, 1),
        out_shape=jax.ShapeDtypeStruct((n, 1), jnp.float32),
    )(*args)


def _gather(table, idx):
    return jnp.take(table, idx, axis=0)


def _segmax(vals, seg, n):
    return jax.ops.segment_max(vals, seg, num_segments=n)


def _bn_fold(st, g, b, cnt, eps=1e-5):
    m = st[0] / cnt
    v = st[1] / cnt - m * m
    s = g * jax.lax.rsqrt(v + eps)
    return s, b - m * s


def kernel(x, edge_index, params):
    p = params
    n = x.shape[0]
    e = edge_index.shape[1]
    src, dst = edge_index[0], edge_index[1]
    cnt = jnp.float32(e)

    def edge_layer(xi, xj, ws, bs, gs, betas, s0=None, t0=None):
        fi = xi.shape[1]
        w1, w2, w3 = ws
        if s0 is not None:
            w1s = w1 * s0[None, :]
            c1 = t0 @ w1.T + bs[0]
        else:
            w1s = w1
            c1 = bs[0]
        a = (w1s[:, :fi] - w1s[:, fi:]).T
        b = w1s[:, fi:].T
        h1, st1 = _mm_first(xi, xj, a, b, c1)
        s1, t1 = _bn_fold(st1, gs[0], betas[0], cnt)
        h2, st2 = _mm_mid(h1, (w2 * s1[None, :]).T, t1 @ w2.T + bs[1])
        s2, t2 = _bn_fold(st2, gs[1], betas[1], cnt)
        h3, st3 = _mm_mid(h2, (w3 * s2[None, :]).T, t2 @ w3.T + bs[2])
        s3, t3 = _bn_fold(st3, gs[2], betas[2], cnt)
        raw = _segmax(h3, dst, n)
        return raw, s3, t3

    xi = _gather(x, dst)
    xj = _gather(x, src)
    st0 = _stats0(xi, xj)
    s0, t0 = _bn_fold(st0, p['m1_bn0_g'], p['m1_bn0_b'], cnt)
    raw1, s13, t13 = edge_layer(
        xi, xj, (p['m1_w1'], p['m1_w2'], p['m1_w3']),
        (p['m1_b1'], p['m1_b2'], p['m1_b3']),
        (p['m1_bn1_g'], p['m1_bn2_g'], p['m1_bn3_g']),
        (p['m1_bn1_b'], p['m1_bn2_b'], p['m1_bn3_b']),
        s0, t0)
    nodes1 = _finalize(raw1, s13, t13)

    raw2, s23, t23 = edge_layer(
        _gather(nodes1, dst), _gather(nodes1, src),
        (p['m2_w1'], p['m2_w2'], p['m2_w3']),
        (p['m2_b1'], p['m2_b2'], p['m2_b3']),
        (p['m2_bn1_g'], p['m2_bn2_g'], p['m2_bn3_g']),
        (p['m2_bn1_b'], p['m2_bn2_b'], p['m2_bn3_b']))

    return _head(raw2, s23, t23, p)
